# baseline (speedup 1.0000x reference)
import functools

import jax
import jax.numpy as jnp
from jax import lax
from jax.experimental import pallas as pl
from jax.experimental.pallas import tpu as pltpu
from jax.experimental.pallas import tpu_sc as plsc

NC = 2
NS = 16
CB = 128
NB_R = 128
NSLOT = 6


def _mlp_body(x_ref, w1_ref, b1_ref, w2_ref, b2_ref, out_ref):
    x = x_ref[...]
    h = lax.dot_general(x, w1_ref[...], (((1,), (1,)), ((), ())),
                        preferred_element_type=jnp.float32)
    h = jnp.maximum(h + b1_ref[...], 0.0)
    h = lax.dot_general(h, w2_ref[...], (((1,), (1,)), ((), ())),
                        preferred_element_type=jnp.float32)
    h = h + b2_ref[...]
    half = h.shape[1] // 2
    out_ref[0] = h[:, :half]
    out_ref[1] = h[:, half:]


def _make_sc_kernel(n_pad, ch, k_steps):
    rpt = n_pad // NS
    nblk = rpt // NB_R
    mesh = plsc.VectorSubcoreMesh(core_axis_name="c", subcore_axis_name="s")

    @functools.partial(
        pl.kernel,
        out_type=jax.ShapeDtypeStruct((NC, n_pad, 64), jnp.float32),
        mesh=mesh,
        compiler_params=pltpu.CompilerParams(use_tc_tiling_on_sc=False),
        scratch_types=[
            pltpu.VMEM_SHARED((n_pad, 32), jnp.int32),
            pltpu.VMEM_SHARED((n_pad, 64), jnp.float32),
            pltpu.VMEM((NSLOT, 2, CB), jnp.int32),
            pltpu.VMEM((rpt, 16), jnp.float32),
            pltpu.VMEM((CB, 64), jnp.float32),
            pltpu.VMEM((CB, 64), jnp.float32),
            pltpu.VMEM((CB, 32), jnp.int32),
            pltpu.VMEM((CB, 32), jnp.int32),
            pltpu.VMEM((CB, 64), jnp.float32),
            pltpu.VMEM((16, 16), jnp.float32),
            pltpu.SemaphoreType.DMA,
            pltpu.SemaphoreType.DMA,
            pltpu.SemaphoreType.DMA,
            pltpu.SemaphoreType.DMA,
            pltpu.SemaphoreType.DMA,
            pltpu.SemaphoreType.DMA,
            pltpu.SemaphoreType.DMA,
            pltpu.SemaphoreType.DMA,
            pltpu.SemaphoreType.DMA,
            pltpu.SemaphoreType.DMA,
        ],
    )
    def prop_kernel(h_hbm, idx_hbm, temp_hbm, out_hbm,
                    g_sh, s_sh, icat, dinv_t, msg_a, msg_b, mbf0, mbf1, wc,
                    temp_t, si0, si1, si2, si3, si4, si5, sg0, sg1, ss0, ss1):
        c = lax.axis_index("c")
        s = lax.axis_index("s")
        base = s * rpt
        pltpu.sync_copy(temp_hbm, temp_t)

        sem_i = (si0, si1, si2, si3, si4, si5)
        sem_g = (sg0, sg1)
        sem_s = (ss0, ss1)
        msgs = (msg_a, msg_b)
        mbfs = (mbf0, mbf1)

        ones = jnp.ones((16,), jnp.float32)
        zeros = jnp.zeros((16,), jnp.float32)
        half = jnp.full((16,), 0.5, jnp.float32)
        threehalf = jnp.full((16,), 1.5, jnp.float32)
        magic = jnp.full((16,), 0x5F3759DF, jnp.int32)
        shift1 = jnp.full((16,), 1, jnp.int32)
        shift16 = jnp.full((16,), 16, jnp.int32)
        rnd = jnp.full((16,), 0x8000, jnp.int32)
        mhi = jnp.full((16,), -65536, jnp.int32)

        def start_idx(j, u):
            pltpu.async_copy(idx_hbm.at[s, j], icat.at[u], sem_i[u])

        def wait_idx(j, u):
            pltpu.make_async_copy(idx_hbm.at[s, j], icat.at[u], sem_i[u]).wait()

        def start_scat(u, p, src=None):
            pltpu.async_copy(msgs[p] if src is None else src,
                             s_sh.at[icat.at[u, 1]], sem_s[p], add=True)

        def wait_scat(u, p, src=None):
            pltpu.make_async_copy(msgs[p] if src is None else src,
                                  s_sh.at[icat.at[u, 1]], sem_s[p]).wait()

        def start_gath(u, p):
            pltpu.async_copy(g_sh.at[icat.at[u, 0]], mbfs[p], sem_g[p])

        def wait_gath(u, p):
            pltpu.make_async_copy(g_sh.at[icat.at[u, 0]], mbfs[p],
                                  sem_g[p]).wait()

        def fill(buf, vec):
            @pl.loop(0, CB)
            def _(r):
                for q in range(4):
                    buf[r, pl.ds(q * 16, 16)] = vec

        def unpack_word(w):
            a = lax.bitcast_convert_type(lax.shift_left(w, shift16),
                                         jnp.float32)
            b = lax.bitcast_convert_type(lax.bitwise_and(w, mhi), jnp.float32)
            return a, b

        def unpack_quads(src, r):
            a0, b0 = unpack_word(src[r, pl.ds(0, 16)])
            a1, b1 = unpack_word(src[r, pl.ds(16, 16)])
            return a0, b0, a1, b1

        def unpack_row(src, dst, r):
            q = unpack_quads(src, r)
            for i in range(4):
                dst[r, pl.ds(i * 16, 16)] = q[i]

        def pack_pair(a, b):
            ia = lax.shift_right_logical(
                lax.add(lax.bitcast_convert_type(a, jnp.int32), rnd), shift16)
            ib = lax.bitwise_and(
                lax.add(lax.bitcast_convert_type(b, jnp.int32), rnd), mhi)
            return lax.bitwise_or(ia, ib)

        def pack_row(dst, r, q0, q1, q2, q3):
            dst[r, pl.ds(0, 16)] = pack_pair(q0, q1)
            dst[r, pl.ds(16, 16)] = pack_pair(q2, q3)

        fill(msg_a, ones)
        fill(msg_b, zeros)

        @pl.loop(0, nblk)
        def _(nb):
            pltpu.sync_copy(msg_b, s_sh.at[pl.ds(base + nb * NB_R, NB_R)])

        plsc.subcore_barrier()

        start_idx(0, 0)
        start_idx(1, 1)
        start_idx(2, 2)
        start_idx(3, 3)

        @pl.loop(0, ch, step=NSLOT)
        def _(jo):
            for u in range(NSLOT):
                j = jo + u
                p = u % 2
                wait_idx(j, u)

                @pl.when(j >= 2)
                def _():
                    wait_scat((u + 4) % NSLOT, p, msg_a)

                start_scat(u, p, msg_a)

                @pl.when(j + 4 < ch)
                def _():
                    start_idx(j + 4, (u + 4) % NSLOT)

        pltpu.make_async_copy(msgs[0], s_sh.at[icat.at[4, 1]], sem_s[0]).wait()
        pltpu.make_async_copy(msgs[0], s_sh.at[icat.at[5, 1]], sem_s[1]).wait()
        plsc.subcore_barrier()

        @pl.loop(0, nblk)
        def _(nb):
            rb = base + nb * NB_R
            pltpu.sync_copy(s_sh.at[pl.ds(rb, NB_R)], msg_a)
            pltpu.sync_copy(h_hbm.at[c, pl.ds(rb, NB_R)], msg_b)
            t0 = temp_t[0]

            @pl.loop(0, NB_R)
            def _(r):
                deg = msg_a[r, pl.ds(0, 16)] + ones
                i32 = lax.bitcast_convert_type(deg, jnp.int32)
                y = lax.bitcast_convert_type(
                    magic - lax.shift_right_arithmetic(i32, shift1),
                    jnp.float32)
                hx = half * deg
                y = y * (threehalf - hx * y * y)
                y = y * (threehalf - hx * y * y)
                y = y * (threehalf - hx * y * y)
                dinv_t[r + nb * NB_R] = y
                gq = []
                for q in range(4):
                    dq = pl.ds(q * 16, 16)
                    hv = msg_b[r, dq]
                    wc[r, dq] = t0 * hv
                    gq.append(y * hv)
                    msg_a[r, dq] = zeros
                pack_row(mbf0, r, *gq)

            pltpu.sync_copy(mbf0, g_sh.at[pl.ds(rb, NB_R)])
            pltpu.sync_copy(wc, out_hbm.at[c, pl.ds(rb, NB_R)])
            pltpu.sync_copy(msg_a, s_sh.at[pl.ds(rb, NB_R)])

        plsc.subcore_barrier()

        @pl.loop(0, k_steps)
        def _(k):
            start_idx(0, 0)
            start_idx(1, 1)
            start_idx(2, 2)
            start_idx(3, 3)
            wait_idx(0, 0)
            start_gath(0, 0)
            wait_idx(1, 1)
            start_gath(1, 1)

            @pl.loop(0, ch, step=NSLOT)
            def _(jo):
                for u in range(NSLOT):
                    j = jo + u
                    p = u % 2
                    wait_gath(u, p)

                    @pl.when(j >= 2)
                    def _():
                        wait_scat((u + 4) % NSLOT, p)

                    @plsc.parallel_loop(0, CB, unroll=8)
                    def _(r):
                        unpack_row(mbfs[p], msgs[p], r)

                    start_scat(u, p)

                    @pl.when(j + 2 < ch)
                    def _():
                        wait_idx(j + 2, (u + 2) % NSLOT)
                        start_gath((u + 2) % NSLOT, p)

                    @pl.when(j + 4 < ch)
                    def _():
                        start_idx(j + 4, (u + 4) % NSLOT)

            wait_scat(4, 0)
            wait_scat(5, 1)
            plsc.subcore_barrier()
            tk = temp_t[k + 1]

            @pl.loop(0, nblk)
            def _(nb):
                rb = base + nb * NB_R
                pltpu.sync_copy(s_sh.at[pl.ds(rb, NB_R)], msg_a)
                pltpu.sync_copy(g_sh.at[pl.ds(rb, NB_R)], mbf1)
                pltpu.sync_copy(out_hbm.at[c, pl.ds(rb, NB_R)], wc)

                @pl.loop(0, NB_R, unroll=4)
                def _(r):
                    dv = dinv_t[r + nb * NB_R]
                    gq = unpack_quads(mbf1, r)
                    gn = []
                    for q in range(4):
                        dq = pl.ds(q * 16, 16)
                        hn = dv * (msg_a[r, dq] + gq[q])
                        wc[r, dq] = wc[r, dq] + tk * hn
                        gn.append(dv * hn)
                        msg_a[r, dq] = zeros
                    pack_row(mbf0, r, *gn)

                pltpu.sync_copy(mbf0, g_sh.at[pl.ds(rb, NB_R)])
                pltpu.sync_copy(wc, out_hbm.at[c, pl.ds(rb, NB_R)])
                pltpu.sync_copy(msg_a, s_sh.at[pl.ds(rb, NB_R)])

            plsc.subcore_barrier()

    return prop_kernel


def kernel(x, edge_index, W1, b1, W2, b2, temp):
    n, d = x.shape
    e = edge_index.shape[1]
    assert d == 128
    n_pad = -(-n // (NS * NB_R)) * (NS * NB_R)

    bn = 1000
    assert n % bn == 0
    h2 = pl.pallas_call(
        _mlp_body,
        grid=(n // bn,),
        in_specs=[
            pl.BlockSpec((bn, d), lambda i: (i, 0)),
            pl.BlockSpec((d, d), lambda i: (0, 0)),
            pl.BlockSpec((1, d), lambda i: (0, 0)),
            pl.BlockSpec((d, d), lambda i: (0, 0)),
            pl.BlockSpec((1, d), lambda i: (0, 0)),
        ],
        out_specs=pl.BlockSpec((NC, bn, 64), lambda i: (0, i, 0)),
        out_shape=jax.ShapeDtypeStruct((NC, n, 64), jnp.float32),
    )(x, W1, b1.reshape(1, d), W2, b2.reshape(1, d))
    h2p = jnp.zeros((NC, n_pad, 64), jnp.float32).at[:, :n].set(h2)

    row = edge_index[0].astype(jnp.int32)
    col = edge_index[1].astype(jnp.int32)
    ch = -(-(-(-e // (NS * CB))) // NSLOT) * NSLOT
    pad = NS * CB * ch - e
    fill = jnp.full((pad,), n, jnp.int32)
    row_p = jnp.concatenate([row, fill]).reshape(NS, ch, CB)
    col_p = jnp.concatenate([col, fill]).reshape(NS, ch, CB)
    idx_p = jnp.stack([row_p, col_p], axis=2)
    kk = temp.shape[0]
    temp_p = jnp.zeros((16, 16), jnp.float32).at[:kk].set(
        jnp.broadcast_to(temp[:, None], (kk, 16)))

    out2 = _make_sc_kernel(n_pad, ch, kk - 1)(h2p, idx_p, temp_p)
    return jnp.concatenate([out2[0, :n], out2[1, :n]], axis=1)


# speedup vs baseline: 1.2048x; 1.2048x over previous
import functools

import jax
import jax.numpy as jnp
from jax import lax
from jax.experimental import pallas as pl
from jax.experimental.pallas import tpu as pltpu
from jax.experimental.pallas import tpu_sc as plsc

NC = 2
NS = 16
CB = 128
NB_R = 128
NSLOT = 8


def _mlp_body(x_ref, w1_ref, b1_ref, w2_ref, b2_ref, out_ref):
    x = x_ref[...]
    h = lax.dot_general(x, w1_ref[...], (((1,), (1,)), ((), ())),
                        preferred_element_type=jnp.float32)
    h = jnp.maximum(h + b1_ref[...], 0.0)
    h = lax.dot_general(h, w2_ref[...], (((1,), (1,)), ((), ())),
                        preferred_element_type=jnp.float32)
    h = h + b2_ref[...]
    half = h.shape[1] // 2
    out_ref[0] = h[:, :half]
    out_ref[1] = h[:, half:]


def _make_sc_kernel(n_pad, ch, k_steps):
    rpt = n_pad // NS
    nblk = rpt // NB_R
    mesh = plsc.VectorSubcoreMesh(core_axis_name="c", subcore_axis_name="s")

    @functools.partial(
        pl.kernel,
        out_type=jax.ShapeDtypeStruct((NC, n_pad, 64), jnp.float32),
        mesh=mesh,
        compiler_params=pltpu.CompilerParams(use_tc_tiling_on_sc=False),
        scratch_types=[
            pltpu.VMEM_SHARED((n_pad, 64), jnp.float32),
            pltpu.VMEM_SHARED((n_pad, 64), jnp.float32),
            pltpu.VMEM((NSLOT, 2, CB), jnp.int32),
            pltpu.VMEM((rpt, 16), jnp.float32),
            pltpu.VMEM((CB, 64), jnp.float32),
            pltpu.VMEM((CB, 64), jnp.float32),
            pltpu.VMEM((CB, 64), jnp.float32),
            pltpu.VMEM((CB, 64), jnp.float32),
            pltpu.VMEM((16, 16), jnp.float32),
            pltpu.SemaphoreType.DMA,
            pltpu.SemaphoreType.DMA,
            pltpu.SemaphoreType.DMA,
            pltpu.SemaphoreType.DMA,
            pltpu.SemaphoreType.DMA,
            pltpu.SemaphoreType.DMA,
            pltpu.SemaphoreType.DMA,
            pltpu.SemaphoreType.DMA,
            pltpu.SemaphoreType.DMA,
            pltpu.SemaphoreType.DMA,
            pltpu.SemaphoreType.DMA,
            pltpu.SemaphoreType.DMA,
            pltpu.SemaphoreType.DMA,
            pltpu.SemaphoreType.DMA,
            pltpu.SemaphoreType.DMA,
            pltpu.SemaphoreType.DMA,
        ],
    )
    def prop_kernel(h_hbm, idx_hbm, temp_hbm, out_hbm,
                    g_sh, s_sh, icat, dinv_t, msg_a, msg_b, msg_c, msg_d,
                    temp_t, si0, si1, si2, si3, si4, si5, si6, si7,
                    sg0, sg1, sg2, sg3, ss0, ss1, ss2, ss3):
        c = lax.axis_index("c")
        s = lax.axis_index("s")
        base = s * rpt
        pltpu.sync_copy(temp_hbm, temp_t)

        sem_i = (si0, si1, si2, si3, si4, si5, si6, si7)
        sem_g = (sg0, sg1, sg2, sg3)
        sem_s = (ss0, ss1, ss2, ss3)
        msgs = (msg_a, msg_b, msg_c, msg_d)

        ones = jnp.ones((16,), jnp.float32)
        zeros = jnp.zeros((16,), jnp.float32)
        half = jnp.full((16,), 0.5, jnp.float32)
        threehalf = jnp.full((16,), 1.5, jnp.float32)
        magic = jnp.full((16,), 0x5F3759DF, jnp.int32)
        shift1 = jnp.full((16,), 1, jnp.int32)

        def start_idx(j, u):
            pltpu.async_copy(idx_hbm.at[s, j], icat.at[u], sem_i[u])

        def wait_idx(j, u):
            pltpu.make_async_copy(idx_hbm.at[s, j], icat.at[u], sem_i[u]).wait()

        def start_scat(u, p, src=None):
            pltpu.async_copy(msgs[p] if src is None else src,
                             s_sh.at[icat.at[u, 1]], sem_s[p], add=True)

        def wait_scat(u, p, src=None):
            pltpu.make_async_copy(msgs[p] if src is None else src,
                                  s_sh.at[icat.at[u, 1]], sem_s[p]).wait()

        def start_gath(u, p):
            pltpu.async_copy(g_sh.at[icat.at[u, 0]], msgs[p], sem_g[p])

        def wait_gath(u, p):
            pltpu.make_async_copy(g_sh.at[icat.at[u, 0]], msgs[p],
                                  sem_g[p]).wait()

        def fill(buf, vec):
            @pl.loop(0, CB, unroll=4)
            def _(r):
                for q in range(4):
                    buf[r, pl.ds(q * 16, 16)] = vec

        fill(msg_a, ones)
        fill(msg_b, zeros)

        @pl.loop(0, nblk)
        def _(nb):
            pltpu.sync_copy(msg_b, s_sh.at[pl.ds(base + nb * NB_R, NB_R)])

        plsc.subcore_barrier()

        start_idx(0, 0)
        start_idx(1, 1)
        start_idx(2, 2)
        start_idx(3, 3)

        @pl.loop(0, ch, step=NSLOT)
        def _(jo):
            for u in range(NSLOT):
                j = jo + u
                p = u % 4
                wait_idx(j, u)

                @pl.when(j >= 4)
                def _():
                    wait_scat((u + 4) % NSLOT, p, msg_a)

                start_scat(u, p, msg_a)

                @pl.when(j + 4 < ch)
                def _():
                    start_idx(j + 4, (u + 4) % NSLOT)

        wait_scat(4, 0, msg_a)
        wait_scat(5, 1, msg_a)
        wait_scat(6, 2, msg_a)
        wait_scat(7, 3, msg_a)
        plsc.subcore_barrier()

        @pl.loop(0, nblk)
        def _(nb):
            rb = base + nb * NB_R
            pltpu.sync_copy(s_sh.at[pl.ds(rb, NB_R)], msg_a)
            pltpu.sync_copy(h_hbm.at[c, pl.ds(rb, NB_R)], msg_b)
            t0 = temp_t[0]

            @pl.loop(0, NB_R, unroll=4)
            def _(r):
                deg = msg_a[r, pl.ds(0, 16)] + ones
                i32 = lax.bitcast_convert_type(deg, jnp.int32)
                y = lax.bitcast_convert_type(
                    magic - lax.shift_right_arithmetic(i32, shift1),
                    jnp.float32)
                hx = half * deg
                y = y * (threehalf - hx * y * y)
                y = y * (threehalf - hx * y * y)
                y = y * (threehalf - hx * y * y)
                dinv_t[r + nb * NB_R] = y
                for q in range(4):
                    dq = pl.ds(q * 16, 16)
                    hv = msg_b[r, dq]
                    msg_c[r, dq] = t0 * hv
                    msg_b[r, dq] = y * hv
                    msg_a[r, dq] = zeros

            pltpu.sync_copy(msg_b, g_sh.at[pl.ds(rb, NB_R)])
            pltpu.sync_copy(msg_c, out_hbm.at[c, pl.ds(rb, NB_R)])
            pltpu.sync_copy(msg_a, s_sh.at[pl.ds(rb, NB_R)])

        plsc.subcore_barrier()

        @pl.loop(0, k_steps)
        def _(k):
            start_idx(0, 0)
            start_idx(1, 1)
            start_idx(2, 2)
            start_idx(3, 3)
            start_idx(4, 4)
            start_idx(5, 5)
            wait_idx(0, 0)
            start_gath(0, 0)
            wait_idx(1, 1)
            start_gath(1, 1)

            @pl.loop(0, ch, step=NSLOT)
            def _(jo):
                for u in range(NSLOT):
                    j = jo + u
                    p = u % 4
                    wait_gath(u, p)

                    @pl.when(j >= 2)
                    def _():
                        wait_scat((u + 6) % NSLOT, (u + 2) % 4)

                    start_scat(u, p)

                    @pl.when(j + 2 < ch)
                    def _():
                        wait_idx(j + 2, (u + 2) % NSLOT)
                        start_gath((u + 2) % NSLOT, (u + 2) % 4)

                    @pl.when(j + 6 < ch)
                    def _():
                        start_idx(j + 6, (u + 6) % NSLOT)

            wait_scat(6, 2)
            wait_scat(7, 3)
            plsc.subcore_barrier()
            tk = temp_t[k + 1]

            @pl.loop(0, nblk)
            def _(nb):
                rb = base + nb * NB_R
                pltpu.sync_copy(s_sh.at[pl.ds(rb, NB_R)], msg_a)
                pltpu.sync_copy(g_sh.at[pl.ds(rb, NB_R)], msg_b)
                pltpu.sync_copy(out_hbm.at[c, pl.ds(rb, NB_R)], msg_c)

                @pl.loop(0, NB_R, unroll=4)
                def _(r):
                    dv = dinv_t[r + nb * NB_R]
                    for q in range(4):
                        dq = pl.ds(q * 16, 16)
                        hn = dv * (msg_a[r, dq] + msg_b[r, dq])
                        msg_c[r, dq] = msg_c[r, dq] + tk * hn
                        msg_b[r, dq] = dv * hn
                        msg_a[r, dq] = zeros

                pltpu.sync_copy(msg_b, g_sh.at[pl.ds(rb, NB_R)])
                pltpu.sync_copy(msg_c, out_hbm.at[c, pl.ds(rb, NB_R)])
                pltpu.sync_copy(msg_a, s_sh.at[pl.ds(rb, NB_R)])

            plsc.subcore_barrier()

    return prop_kernel


def kernel(x, edge_index, W1, b1, W2, b2, temp):
    n, d = x.shape
    e = edge_index.shape[1]
    assert d == 128
    n_pad = -(-n // (NS * NB_R)) * (NS * NB_R)

    bn = 1000
    assert n % bn == 0
    h2 = pl.pallas_call(
        _mlp_body,
        grid=(n // bn,),
        in_specs=[
            pl.BlockSpec((bn, d), lambda i: (i, 0)),
            pl.BlockSpec((d, d), lambda i: (0, 0)),
            pl.BlockSpec((1, d), lambda i: (0, 0)),
            pl.BlockSpec((d, d), lambda i: (0, 0)),
            pl.BlockSpec((1, d), lambda i: (0, 0)),
        ],
        out_specs=pl.BlockSpec((NC, bn, 64), lambda i: (0, i, 0)),
        out_shape=jax.ShapeDtypeStruct((NC, n, 64), jnp.float32),
    )(x, W1, b1.reshape(1, d), W2, b2.reshape(1, d))
    h2p = jnp.zeros((NC, n_pad, 64), jnp.float32).at[:, :n].set(h2)

    row = edge_index[0].astype(jnp.int32)
    col = edge_index[1].astype(jnp.int32)
    ch = -(-(-(-e // (NS * CB))) // NSLOT) * NSLOT
    pad = NS * CB * ch - e
    fill = jnp.full((pad,), n, jnp.int32)
    row_p = jnp.concatenate([row, fill]).reshape(NS, ch, CB)
    col_p = jnp.concatenate([col, fill]).reshape(NS, ch, CB)
    idx_p = jnp.stack([row_p, col_p], axis=2)
    kk = temp.shape[0]
    temp_p = jnp.zeros((16, 16), jnp.float32).at[:kk].set(
        jnp.broadcast_to(temp[:, None], (kk, 16)))

    out2 = _make_sc_kernel(n_pad, ch, kk - 1)(h2p, idx_p, temp_p)
    return jnp.concatenate([out2[0, :n], out2[1, :n]], axis=1)


# speedup vs baseline: 1.2458x; 1.0340x over previous
import functools

import jax
import jax.numpy as jnp
from jax import lax
from jax.experimental import pallas as pl
from jax.experimental.pallas import tpu as pltpu
from jax.experimental.pallas import tpu_sc as plsc

NC = 2
NS = 16
CB = 128
NB_R = 128
NSLOT = 8


def _mlp_body(x_ref, w1_ref, b1_ref, w2_ref, b2_ref, out_ref):
    x = x_ref[...]
    h = lax.dot_general(x, w1_ref[...], (((1,), (1,)), ((), ())),
                        preferred_element_type=jnp.float32)
    h = jnp.maximum(h + b1_ref[...], 0.0)
    h = lax.dot_general(h, w2_ref[...], (((1,), (1,)), ((), ())),
                        preferred_element_type=jnp.float32)
    h = h + b2_ref[...]
    half = h.shape[1] // 2
    out_ref[0] = h[:, :half]
    out_ref[1] = h[:, half:]


def _make_sc_kernel(n_pad, ch, k_steps):
    rpt = n_pad // NS
    nblk = rpt // NB_R
    mesh = plsc.VectorSubcoreMesh(core_axis_name="c", subcore_axis_name="s")

    @functools.partial(
        pl.kernel,
        out_type=jax.ShapeDtypeStruct((NC, n_pad, 64), jnp.float32),
        mesh=mesh,
        compiler_params=pltpu.CompilerParams(use_tc_tiling_on_sc=False),
        scratch_types=[
            pltpu.VMEM_SHARED((n_pad, 64), jnp.float32),
            pltpu.VMEM_SHARED((n_pad, 64), jnp.float32),
            pltpu.VMEM((NSLOT, 2, CB), jnp.int32),
            pltpu.VMEM((rpt, 16), jnp.float32),
            pltpu.VMEM((CB, 64), jnp.float32),
            pltpu.VMEM((CB, 64), jnp.float32),
            pltpu.VMEM((CB, 64), jnp.float32),
            pltpu.VMEM((CB, 64), jnp.float32),
            pltpu.VMEM((16, 16), jnp.float32),
            pltpu.SemaphoreType.DMA,
            pltpu.SemaphoreType.DMA,
            pltpu.SemaphoreType.DMA,
            pltpu.SemaphoreType.DMA,
            pltpu.SemaphoreType.DMA,
            pltpu.SemaphoreType.DMA,
            pltpu.SemaphoreType.DMA,
            pltpu.SemaphoreType.DMA,
            pltpu.SemaphoreType.DMA,
            pltpu.SemaphoreType.DMA,
            pltpu.SemaphoreType.DMA,
            pltpu.SemaphoreType.DMA,
            pltpu.SemaphoreType.DMA,
            pltpu.SemaphoreType.DMA,
            pltpu.SemaphoreType.DMA,
            pltpu.SemaphoreType.DMA,
        ],
    )
    def prop_kernel(h_hbm, idx_hbm, temp_hbm, out_hbm,
                    g_sh, s_sh, icat, dinv_t, msg_a, msg_b, msg_c, msg_d,
                    temp_t, si0, si1, si2, si3, si4, si5, si6, si7,
                    sg0, sg1, sg2, sg3, ss0, ss1, ss2, ss3):
        c = lax.axis_index("c")
        s = lax.axis_index("s")
        base = s * rpt
        pltpu.sync_copy(temp_hbm, temp_t)

        sem_i = (si0, si1, si2, si3, si4, si5, si6, si7)
        sem_g = (sg0, sg1, sg2, sg3)
        sem_s = (ss0, ss1, ss2, ss3)
        msgs = (msg_a, msg_b, msg_c, msg_d)

        ones = jnp.ones((16,), jnp.float32)
        zeros = jnp.zeros((16,), jnp.float32)
        half = jnp.full((16,), 0.5, jnp.float32)
        threehalf = jnp.full((16,), 1.5, jnp.float32)
        magic = jnp.full((16,), 0x5F3759DF, jnp.int32)
        shift1 = jnp.full((16,), 1, jnp.int32)

        def start_idx(j, u):
            pltpu.async_copy(idx_hbm.at[s, j], icat.at[u], sem_i[u])

        def wait_idx(j, u):
            pltpu.make_async_copy(idx_hbm.at[s, j], icat.at[u], sem_i[u]).wait()

        def start_scat(u, p, src=None):
            pltpu.async_copy(msgs[p] if src is None else src,
                             s_sh.at[icat.at[u, 1]], sem_s[p], add=True)

        def wait_scat(u, p, src=None):
            pltpu.make_async_copy(msgs[p] if src is None else src,
                                  s_sh.at[icat.at[u, 1]], sem_s[p]).wait()

        def start_gath(u, p):
            pltpu.async_copy(g_sh.at[icat.at[u, 0]], msgs[p], sem_g[p])

        def wait_gath(u, p):
            pltpu.make_async_copy(g_sh.at[icat.at[u, 0]], msgs[p],
                                  sem_g[p]).wait()

        def fill(buf, vec):
            @pl.loop(0, CB, unroll=4)
            def _(r):
                for q in range(4):
                    buf[r, pl.ds(q * 16, 16)] = vec

        fill(msg_a, ones)
        fill(msg_b, zeros)

        @pl.loop(0, nblk)
        def _(nb):
            pltpu.sync_copy(msg_b, s_sh.at[pl.ds(base + nb * NB_R, NB_R)])

        plsc.subcore_barrier()

        start_idx(0, 0)
        start_idx(1, 1)
        start_idx(2, 2)
        start_idx(3, 3)

        @pl.loop(0, ch, step=NSLOT)
        def _(jo):
            for u in range(NSLOT):
                j = jo + u
                p = u % 4
                wait_idx(j, u)

                @pl.when(j >= 4)
                def _():
                    wait_scat((u + 4) % NSLOT, p, msg_a)

                start_scat(u, p, msg_a)

                @pl.when(j + 4 < ch)
                def _():
                    start_idx(j + 4, (u + 4) % NSLOT)

        wait_scat(4, 0, msg_a)
        wait_scat(5, 1, msg_a)
        wait_scat(6, 2, msg_a)
        wait_scat(7, 3, msg_a)
        plsc.subcore_barrier()

        @pl.loop(0, nblk)
        def _(nb):
            rb = base + nb * NB_R
            ina = pltpu.async_copy(s_sh.at[pl.ds(rb, NB_R)], msg_a, sg0)
            inb = pltpu.async_copy(h_hbm.at[c, pl.ds(rb, NB_R)], msg_b, sg1)
            ina.wait()
            inb.wait()
            t0 = temp_t[0]

            @pl.loop(0, NB_R, unroll=4)
            def _(r):
                deg = msg_a[r, pl.ds(0, 16)] + ones
                i32 = lax.bitcast_convert_type(deg, jnp.int32)
                y = lax.bitcast_convert_type(
                    magic - lax.shift_right_arithmetic(i32, shift1),
                    jnp.float32)
                hx = half * deg
                y = y * (threehalf - hx * y * y)
                y = y * (threehalf - hx * y * y)
                y = y * (threehalf - hx * y * y)
                dinv_t[r + nb * NB_R] = y
                for q in range(4):
                    dq = pl.ds(q * 16, 16)
                    hv = msg_b[r, dq]
                    msg_c[r, dq] = t0 * hv
                    msg_b[r, dq] = y * hv
                    msg_a[r, dq] = zeros

            oa = pltpu.async_copy(msg_b, g_sh.at[pl.ds(rb, NB_R)], ss0)
            ob = pltpu.async_copy(msg_c, out_hbm.at[c, pl.ds(rb, NB_R)], ss1)
            oc = pltpu.async_copy(msg_a, s_sh.at[pl.ds(rb, NB_R)], ss2)
            oa.wait()
            ob.wait()
            oc.wait()

        plsc.subcore_barrier()

        @pl.loop(0, k_steps)
        def _(k):
            start_idx(0, 0)
            start_idx(1, 1)
            start_idx(2, 2)
            start_idx(3, 3)
            start_idx(4, 4)
            start_idx(5, 5)
            wait_idx(0, 0)
            start_gath(0, 0)
            wait_idx(1, 1)
            start_gath(1, 1)

            @pl.loop(0, ch, step=NSLOT)
            def _(jo):
                for u in range(NSLOT):
                    j = jo + u
                    p = u % 4
                    wait_gath(u, p)

                    @pl.when(j >= 2)
                    def _():
                        wait_scat((u + 6) % NSLOT, (u + 2) % 4)

                    start_scat(u, p)

                    @pl.when(j + 2 < ch)
                    def _():
                        wait_idx(j + 2, (u + 2) % NSLOT)
                        start_gath((u + 2) % NSLOT, (u + 2) % 4)

                    @pl.when(j + 6 < ch)
                    def _():
                        start_idx(j + 6, (u + 6) % NSLOT)

            wait_scat(6, 2)
            wait_scat(7, 3)
            plsc.subcore_barrier()
            tk = temp_t[k + 1]

            @pl.loop(0, nblk)
            def _(nb):
                rb = base + nb * NB_R
                ina = pltpu.async_copy(s_sh.at[pl.ds(rb, NB_R)], msg_a, sg0)
                inb = pltpu.async_copy(g_sh.at[pl.ds(rb, NB_R)], msg_b, sg1)
                inc = pltpu.async_copy(out_hbm.at[c, pl.ds(rb, NB_R)],
                                       msg_c, sg2)
                ina.wait()
                inb.wait()
                inc.wait()

                @pl.loop(0, NB_R, unroll=4)
                def _(r):
                    dv = dinv_t[r + nb * NB_R]
                    for q in range(4):
                        dq = pl.ds(q * 16, 16)
                        hn = dv * (msg_a[r, dq] + msg_b[r, dq])
                        msg_c[r, dq] = msg_c[r, dq] + tk * hn
                        msg_b[r, dq] = dv * hn
                        msg_a[r, dq] = zeros

                oa = pltpu.async_copy(msg_b, g_sh.at[pl.ds(rb, NB_R)], ss0)
                ob = pltpu.async_copy(msg_c, out_hbm.at[c, pl.ds(rb, NB_R)],
                                      ss1)
                oc = pltpu.async_copy(msg_a, s_sh.at[pl.ds(rb, NB_R)], ss2)
                oa.wait()
                ob.wait()
                oc.wait()

            plsc.subcore_barrier()

    return prop_kernel


def kernel(x, edge_index, W1, b1, W2, b2, temp):
    n, d = x.shape
    e = edge_index.shape[1]
    assert d == 128
    n_pad = -(-n // (NS * NB_R)) * (NS * NB_R)

    bn = 1000
    assert n % bn == 0
    h2 = pl.pallas_call(
        _mlp_body,
        grid=(n // bn,),
        in_specs=[
            pl.BlockSpec((bn, d), lambda i: (i, 0)),
            pl.BlockSpec((d, d), lambda i: (0, 0)),
            pl.BlockSpec((1, d), lambda i: (0, 0)),
            pl.BlockSpec((d, d), lambda i: (0, 0)),
            pl.BlockSpec((1, d), lambda i: (0, 0)),
        ],
        out_specs=pl.BlockSpec((NC, bn, 64), lambda i: (0, i, 0)),
        out_shape=jax.ShapeDtypeStruct((NC, n_pad, 64), jnp.float32),
    )(x, W1, b1.reshape(1, d), W2, b2.reshape(1, d))

    row = edge_index[0].astype(jnp.int32)
    col = edge_index[1].astype(jnp.int32)
    ch = -(-(-(-e // (NS * CB))) // NSLOT) * NSLOT
    pad = NS * CB * ch - e
    fill = jnp.full((pad,), n, jnp.int32)
    row_p = jnp.concatenate([row, fill]).reshape(NS, ch, CB)
    col_p = jnp.concatenate([col, fill]).reshape(NS, ch, CB)
    idx_p = jnp.stack([row_p, col_p], axis=2)
    kk = temp.shape[0]
    temp_p = jnp.zeros((16, 16), jnp.float32).at[:kk].set(
        jnp.broadcast_to(temp[:, None], (kk, 16)))

    out2 = _make_sc_kernel(n_pad, ch, kk - 1)(h2, idx_p, temp_p)
    return jnp.concatenate([out2[0, :n], out2[1, :n]], axis=1)
